# Initial kernel scaffold; baseline (speedup 1.0000x reference)
import jax
import jax.numpy as jnp
from jax.experimental import pallas as pl


def kernel(x, edge_index, W1l, b1l, W1r, W2l, b2l, W2r):
    raise NotImplementedError("write your pallas kernel here")


# speedup vs baseline: 7.4776x; 7.4776x over previous
import functools

import jax
import jax.numpy as jnp
from jax import lax
from jax.experimental import pallas as pl
from jax.experimental.pallas import tpu as pltpu
from jax.experimental.pallas import tpu_sc as plsc

N = 10000
E = 320000
D = 128
NC, NS = 2, 16
NW = NC * NS
EPT = E // NW
G = 80
NCH = EPT // G
NP = 10240
RPT = NP // NS

_MESH = plsc.VectorSubcoreMesh(
    core_axis_name="c", subcore_axis_name="s", num_cores=NC, num_subcores=NS)


def _seg_body(with_deg, feat, srcs, dsts, zf, zd, ones, out, deg_out,
              src_v, dst_v, rows, acc, sem, ones_v, dacc):
  cid = lax.axis_index("c")
  sid = lax.axis_index("s")
  wid = cid * NS + sid

  pltpu.sync_copy(zf, acc.at[pl.ds(sid * RPT, RPT)])
  if with_deg:
    @pl.when(sid == 0)
    def _():
      pltpu.sync_copy(zd, dacc)
    pltpu.sync_copy(ones, ones_v)
  pltpu.sync_copy(srcs.at[wid], src_v)
  pltpu.sync_copy(dsts.at[wid], dst_v)
  plsc.subcore_barrier()

  def step(j, carry):
    pltpu.async_copy(feat.at[src_v.at[j]], rows, sem).wait()
    pltpu.sync_copy(rows, acc.at[dst_v.at[j]], add=True)
    if with_deg:
      pltpu.sync_copy(ones_v, dacc.at[dst_v.at[j]], add=True)
    return carry

  lax.fori_loop(0, NCH, step, 0)
  plsc.subcore_barrier()

  pltpu.sync_copy(acc.at[pl.ds(sid * RPT, RPT)],
                  out.at[cid, pl.ds(sid * RPT, RPT)])
  if with_deg:
    @pl.when(sid == 0)
    def _():
      pltpu.sync_copy(dacc, deg_out.at[cid])


def _make_seg(with_deg):
  out_type = [jax.ShapeDtypeStruct((NC, NP, D), jnp.float32)]
  if with_deg:
    out_type.append(jax.ShapeDtypeStruct((NC, N), jnp.float32))
  scratch = [
      pltpu.VMEM((NCH, G), jnp.int32),
      pltpu.VMEM((NCH, G), jnp.int32),
      pltpu.VMEM((G, D), jnp.float32),
      pltpu.VMEM_SHARED((NP, D), jnp.float32),
      pltpu.SemaphoreType.DMA,
      pltpu.VMEM((G,), jnp.float32) if with_deg else None,
      pltpu.VMEM_SHARED((N,), jnp.float32) if with_deg else None,
  ]
  scratch = [s for s in scratch if s is not None]

  if with_deg:
    def body(feat, srcs, dsts, zf, zd, ones, out, deg_out,
             src_v, dst_v, rows, acc, sem, ones_v, dacc):
      _seg_body(True, feat, srcs, dsts, zf, zd, ones, out, deg_out,
                src_v, dst_v, rows, acc, sem, ones_v, dacc)
  else:
    def body(feat, srcs, dsts, zf, out,
             src_v, dst_v, rows, acc, sem):
      _seg_body(False, feat, srcs, dsts, zf, None, None, out, None,
                src_v, dst_v, rows, acc, sem, None, None)

  return pl.kernel(body, out_type=out_type, mesh=_MESH, scratch_types=scratch)


_seg_sum_deg = _make_seg(True)
_seg_sum = _make_seg(False)

R = 400


def _dense1_body(acc_ref, deg_ref, x_ref, wl_ref, bl_ref, wr_ref,
                 h_ref, dc_ref):
  a = acc_ref[0] + acc_ref[1]
  d = deg_ref[0] + deg_ref[1]
  dc = jnp.maximum(d, 1.0)
  mean = a / dc
  hl = lax.dot_general(mean, wl_ref[...], (((1,), (1,)), ((), ())),
                       preferred_element_type=jnp.float32)
  hr = lax.dot_general(x_ref[...], wr_ref[...], (((1,), (1,)), ((), ())),
                       preferred_element_type=jnp.float32)
  h_ref[...] = jnp.maximum(hl + bl_ref[0] + hr, 0.0)
  dc_ref[...] = dc


def _dense2_body(acc_ref, dc_ref, h_ref, wl_ref, bl_ref, wr_ref, out_ref):
  a = acc_ref[0] + acc_ref[1]
  mean = a / dc_ref[...]
  ol = lax.dot_general(mean, wl_ref[...], (((1,), (1,)), ((), ())),
                       preferred_element_type=jnp.float32)
  orr = lax.dot_general(h_ref[...], wr_ref[...], (((1,), (1,)), ((), ())),
                        preferred_element_type=jnp.float32)
  out_ref[...] = ol + bl_ref[0] + orr


_W_SPEC = pl.BlockSpec((D, D), lambda i: (0, 0))
_B_SPEC = pl.BlockSpec((1, D), lambda i: (0, 0))
_ROW_SPEC = pl.BlockSpec((R, D), lambda i: (i, 0))
_ACC_SPEC = pl.BlockSpec((NC, R, D), lambda i: (0, i, 0))
_DEG_SPEC = pl.BlockSpec((NC, R, 1), lambda i: (0, i, 0))
_DC_SPEC = pl.BlockSpec((R, 1), lambda i: (i, 0))

_dense1 = pl.pallas_call(
    _dense1_body,
    grid=(N // R,),
    in_specs=[_ACC_SPEC, _DEG_SPEC, _ROW_SPEC, _W_SPEC, _B_SPEC, _W_SPEC],
    out_specs=[_ROW_SPEC, _DC_SPEC],
    out_shape=[jax.ShapeDtypeStruct((N, D), jnp.float32),
               jax.ShapeDtypeStruct((N, 1), jnp.float32)],
)

_dense2 = pl.pallas_call(
    _dense2_body,
    grid=(N // R,),
    in_specs=[_ACC_SPEC, _DC_SPEC, _ROW_SPEC, _W_SPEC, _B_SPEC, _W_SPEC],
    out_specs=_ROW_SPEC,
    out_shape=jax.ShapeDtypeStruct((N, D), jnp.float32),
)


def kernel(x, edge_index, W1l, b1l, W1r, W2l, b2l, W2r):
  ei = edge_index.astype(jnp.int32)
  src = ei[0].reshape(NW, NCH, G)
  dst = ei[1].reshape(NW, NCH, G)
  zf = jnp.zeros((RPT, D), jnp.float32)
  zd = jnp.zeros((N,), jnp.float32)
  ones = jnp.ones((G,), jnp.float32)

  acc1, deg = _seg_sum_deg(x, src, dst, zf, zd, ones)
  h, dclip = _dense1(acc1, deg.reshape(NC, N, 1), x, W1l, b1l.reshape(1, D), W1r)
  acc2, = _seg_sum(h, src, dst, zf)
  out = _dense2(acc2, dclip, h, W2l, b2l.reshape(1, D), W2r)
  return out


# speedup vs baseline: 11.5638x; 1.5465x over previous
import functools

import jax
import jax.numpy as jnp
from jax import lax
from jax.experimental import pallas as pl
from jax.experimental.pallas import tpu as pltpu
from jax.experimental.pallas import tpu_sc as plsc

N = 10000
E = 320000
D = 128
NC, NS = 2, 16
NW = NC * NS
EPT = E // NW
G = 80
NCH = EPT // G
NP = 10240
RPT = NP // NS

_MESH = plsc.VectorSubcoreMesh(
    core_axis_name="c", subcore_axis_name="s", num_cores=NC, num_subcores=NS)


def _seg_body(with_deg, feat, srcs, dsts, zf, zd, ones, out, deg_out,
              src_v, dst_v, rows0, rows1, acc, sem0, sem1, ones_v, dacc):
  cid = lax.axis_index("c")
  sid = lax.axis_index("s")
  wid = cid * NS + sid

  pltpu.sync_copy(zf, acc.at[pl.ds(sid * RPT, RPT)])
  if with_deg:
    @pl.when(sid == 0)
    def _():
      pltpu.sync_copy(zd, dacc)
    pltpu.sync_copy(ones, ones_v)
  pltpu.sync_copy(srcs.at[pl.ds(wid * EPT, EPT)], src_v)
  pltpu.sync_copy(dsts.at[wid], dst_v)
  plsc.subcore_barrier()

  def gstart(j, rows, sem):
    off = pl.multiple_of(j * G, 8)
    pltpu.async_copy(feat.at[src_v.at[pl.ds(off, G)]], rows, sem)

  def gwait(rows, sem):
    pltpu.make_async_copy(feat.at[src_v.at[pl.ds(0, G)]], rows, sem).wait()

  def scat(j, rows):
    pltpu.sync_copy(rows, acc.at[dst_v.at[j]], add=True)
    if with_deg:
      pltpu.sync_copy(ones_v, dacc.at[dst_v.at[j]], add=True)

  gstart(0, rows0, sem0)

  def pair(i, carry):
    j0 = 2 * i
    gstart(j0 + 1, rows1, sem1)
    gwait(rows0, sem0)
    scat(j0, rows0)
    gstart(j0 + 2, rows0, sem0)
    gwait(rows1, sem1)
    scat(j0 + 1, rows1)
    return carry

  lax.fori_loop(0, (NCH - 1) // 2, pair, 0)
  gwait(rows0, sem0)
  scat(NCH - 1, rows0)
  plsc.subcore_barrier()

  pltpu.sync_copy(acc.at[pl.ds(sid * RPT, RPT)],
                  out.at[cid, pl.ds(sid * RPT, RPT)])
  if with_deg:
    @pl.when(sid == 0)
    def _():
      pltpu.sync_copy(dacc, deg_out.at[cid])


def _make_seg(with_deg):
  out_type = [jax.ShapeDtypeStruct((NC, NP, D), jnp.float32)]
  if with_deg:
    out_type.append(jax.ShapeDtypeStruct((NC, N), jnp.float32))
  scratch = [
      pltpu.VMEM((EPT,), jnp.int32),
      pltpu.VMEM((NCH, G), jnp.int32),
      pltpu.VMEM((G, D), jnp.float32),
      pltpu.VMEM((G, D), jnp.float32),
      pltpu.VMEM_SHARED((NP, D), jnp.float32),
      pltpu.SemaphoreType.DMA,
      pltpu.SemaphoreType.DMA,
      pltpu.VMEM((G,), jnp.float32) if with_deg else None,
      pltpu.VMEM_SHARED((N,), jnp.float32) if with_deg else None,
  ]
  scratch = [s for s in scratch if s is not None]

  if with_deg:
    def body(feat, srcs, dsts, zf, zd, ones, out, deg_out,
             src_v, dst_v, rows0, rows1, acc, sem0, sem1, ones_v, dacc):
      _seg_body(True, feat, srcs, dsts, zf, zd, ones, out, deg_out,
                src_v, dst_v, rows0, rows1, acc, sem0, sem1, ones_v, dacc)
  else:
    def body(feat, srcs, dsts, zf, out,
             src_v, dst_v, rows0, rows1, acc, sem0, sem1):
      _seg_body(False, feat, srcs, dsts, zf, None, None, out, None,
                src_v, dst_v, rows0, rows1, acc, sem0, sem1, None, None)

  return pl.kernel(body, out_type=out_type, mesh=_MESH, scratch_types=scratch)


_seg_sum_deg = _make_seg(True)
_seg_sum = _make_seg(False)

R = 400


def _dense1_body(acc_ref, deg_ref, x_ref, wl_ref, bl_ref, wr_ref,
                 h_ref, dc_ref):
  a = acc_ref[0] + acc_ref[1]
  d = deg_ref[0] + deg_ref[1]
  dc = jnp.maximum(d, 1.0)
  mean = a / dc
  hl = lax.dot_general(mean, wl_ref[...], (((1,), (1,)), ((), ())),
                       preferred_element_type=jnp.float32)
  hr = lax.dot_general(x_ref[...], wr_ref[...], (((1,), (1,)), ((), ())),
                       preferred_element_type=jnp.float32)
  h_ref[...] = jnp.maximum(hl + bl_ref[0] + hr, 0.0)
  dc_ref[...] = dc


def _dense2_body(acc_ref, dc_ref, h_ref, wl_ref, bl_ref, wr_ref, out_ref):
  a = acc_ref[0] + acc_ref[1]
  mean = a / dc_ref[...]
  ol = lax.dot_general(mean, wl_ref[...], (((1,), (1,)), ((), ())),
                       preferred_element_type=jnp.float32)
  orr = lax.dot_general(h_ref[...], wr_ref[...], (((1,), (1,)), ((), ())),
                        preferred_element_type=jnp.float32)
  out_ref[...] = ol + bl_ref[0] + orr


_W_SPEC = pl.BlockSpec((D, D), lambda i: (0, 0))
_B_SPEC = pl.BlockSpec((1, D), lambda i: (0, 0))
_ROW_SPEC = pl.BlockSpec((R, D), lambda i: (i, 0))
_ACC_SPEC = pl.BlockSpec((NC, R, D), lambda i: (0, i, 0))
_DEG_SPEC = pl.BlockSpec((NC, R, 1), lambda i: (0, i, 0))
_DC_SPEC = pl.BlockSpec((R, 1), lambda i: (i, 0))

_dense1 = pl.pallas_call(
    _dense1_body,
    grid=(N // R,),
    in_specs=[_ACC_SPEC, _DEG_SPEC, _ROW_SPEC, _W_SPEC, _B_SPEC, _W_SPEC],
    out_specs=[_ROW_SPEC, _DC_SPEC],
    out_shape=[jax.ShapeDtypeStruct((N, D), jnp.float32),
               jax.ShapeDtypeStruct((N, 1), jnp.float32)],
)

_dense2 = pl.pallas_call(
    _dense2_body,
    grid=(N // R,),
    in_specs=[_ACC_SPEC, _DC_SPEC, _ROW_SPEC, _W_SPEC, _B_SPEC, _W_SPEC],
    out_specs=_ROW_SPEC,
    out_shape=jax.ShapeDtypeStruct((N, D), jnp.float32),
)


def kernel(x, edge_index, W1l, b1l, W1r, W2l, b2l, W2r):
  ei = edge_index.astype(jnp.int32)
  src = ei[0]
  dst = ei[1].reshape(NW, NCH, G)
  zf = jnp.zeros((RPT, D), jnp.float32)
  zd = jnp.zeros((N,), jnp.float32)
  ones = jnp.ones((G,), jnp.float32)

  acc1, deg = _seg_sum_deg(x, src, dst, zf, zd, ones)
  h, dclip = _dense1(acc1, deg.reshape(NC, N, 1), x, W1l, b1l.reshape(1, D), W1r)
  acc2, = _seg_sum(h, src, dst, zf)
  out = _dense2(acc2, dclip, h, W2l, b2l.reshape(1, D), W2r)
  return out
